# Initial kernel scaffold; baseline (speedup 1.0000x reference)
import jax
import jax.numpy as jnp
from jax.experimental import pallas as pl


def kernel(H, Z, block_id, batch_id, sigma, W1, b1, W2, b2, gamma, beta):
    raise NotImplementedError("write your pallas kernel here")


# speedup vs baseline: 27.7742x; 27.7742x over previous
import functools

import jax
import jax.numpy as jnp
from jax import lax
from jax.experimental import pallas as pl
from jax.experimental.pallas import tpu as pltpu

N = 65536
N_BLOCKS = 4096
N_BATCH = 16
D_HIDDEN = 128
N_CHANNEL = 14
N_RBF = 16
CUTOFF = 7.0
D_Z = N_CHANNEL * 3

_BLK1 = 8192
_BLK2 = 4096

_DN_T0 = (((0,), (0,)), ((), ()))
_DN_STD = (((1,), (0,)), ((), ()))


def _dot(a, b, dn):
    return lax.dot_general(a, b, dn, precision=lax.Precision.HIGHEST,
                           preferred_element_type=jnp.float32)


def _segment_bounds(batch_col):
    b_io = lax.broadcasted_iota(jnp.int32, (1, N_BATCH), 1)
    lt = (batch_col < b_io).astype(jnp.int32)
    le = (batch_col <= b_io).astype(jnp.int32)
    lo = jnp.sum(lt, axis=0, keepdims=True)
    hi = jnp.sum(le, axis=0, keepdims=True)
    return lo, hi


def _onehot(bi_col, lo, hi):
    oh = jnp.logical_and(bi_col >= lo, bi_col < hi)
    return jnp.where(oh, jnp.float32(1.0), jnp.float32(0.0))


def _silu(x):
    return x / (1.0 + jnp.exp(-x))


def _stats_kernel(z_ref, bi_ref, batch_ref, sigma_ref, w1t_ref, b1_ref,
                  w2t_ref, b2_ref,
                  zc_ref, rrep_ref, h2_ref, resc_ref,
                  s1_acc, s2_acc, cnt_acc, lo_acc, hi_acc, *, ngrid):
    i = pl.program_id(0)

    @pl.when(i == 0)
    def _init():
        lo, hi = _segment_bounds(batch_ref[...])
        lo_acc[...] = lo
        hi_acc[...] = hi
        s1_acc[...] = jnp.zeros_like(s1_acc)
        s2_acc[...] = jnp.zeros_like(s2_acc)
        cnt_acc[...] = jnp.zeros_like(cnt_acc)

    z = z_ref[...]
    oh = _onehot(bi_ref[...], lo_acc[...], hi_acc[...])
    s1_acc[...] += _dot(oh, z, _DN_T0)
    s2_acc[...] += _dot(oh, z * z, _DN_T0)
    cnt_acc[...] += _dot(oh, jnp.ones((oh.shape[0], 1), jnp.float32), _DN_T0)

    @pl.when(i == ngrid - 1)
    def _middle():
        s1 = s1_acc[...]
        s2 = s2_acc[...]
        cnt = cnt_acc[...]
        cntc = jnp.maximum(cnt, 1.0)
        zc = s1 / cntc
        r_io = lax.broadcasted_iota(jnp.int32, (D_Z, N_CHANNEL), 0)
        c_io = lax.broadcasted_iota(jnp.int32, (D_Z, N_CHANNEL), 1)
        grp = jnp.where(jnp.logical_and(r_io >= 3 * c_io, r_io < 3 * c_io + 3),
                        jnp.float32(1.0), jnp.float32(0.0))
        t2 = _dot(s2 - s1 * s1 / cntc, grp, _DN_STD)
        denom = jnp.maximum(3.0 * cnt - 1.0, 1.0)
        std = jnp.sqrt(t2 / denom)
        resc = sigma_ref[...] / std
        resc_ref[...] = resc
        resc_safe = jnp.minimum(resc, jnp.float32(1e30))
        ce = lax.broadcasted_iota(jnp.int32, (N_CHANNEL, N_CHANNEL * N_RBF), 0)
        je = lax.broadcasted_iota(jnp.int32, (N_CHANNEL, N_CHANNEL * N_RBF), 1)
        exp_m = jnp.where(
            jnp.logical_and(je >= N_RBF * ce, je < N_RBF * ce + N_RBF),
            jnp.float32(1.0), jnp.float32(0.0))
        dsc = _dot(resc_safe / CUTOFF, exp_m, _DN_STD)
        p = 5.0
        ea = -(p + 1.0) * (p + 2.0) / 2.0
        eb = p * (p + 2.0)
        ec = -p * (p + 1.0) / 2.0
        d4 = (dsc * dsc) * (dsc * dsc)
        env = 1.0 / dsc + ea * d4 + eb * d4 * dsc + ec * d4 * dsc * dsc
        env = jnp.where(dsc < 1.0, env, 0.0)
        j_io = lax.broadcasted_iota(jnp.int32, (1, N_CHANNEL * N_RBF), 1)
        step = jnp.float32(1.0 / (N_RBF - 1))
        offs = (j_io % N_RBF).astype(jnp.float32) * step
        coeff = -0.5 / (step * step)
        gauss = jnp.exp(coeff * (dsc - offs) * (dsc - offs))
        rbf = env * gauss
        h1 = _silu(_dot(rbf, w1t_ref[...], _DN_STD) + b1_ref[...])
        h2 = _silu(_dot(h1, w2t_ref[...], _DN_STD) + b2_ref[...])
        h2_ref[...] = h2
        c3 = lax.broadcasted_iota(jnp.int32, (N_CHANNEL, D_Z), 0)
        j3 = lax.broadcasted_iota(jnp.int32, (N_CHANNEL, D_Z), 1)
        rep3 = jnp.where(jnp.logical_and(j3 >= 3 * c3, j3 < 3 * c3 + 3),
                         jnp.float32(1.0), jnp.float32(0.0))
        rrep_ref[...] = _dot(resc_safe, rep3, _DN_STD)
        zc_ref[...] = zc


def _apply_kernel(h_ref, z_ref, bi_ref, batch_ref, zc_ref, rrep_ref,
                  hb_ref, gamma_ref, beta_ref,
                  ho_ref, zo_ref, lo_acc, hi_acc):
    i = pl.program_id(0)

    @pl.when(i == 0)
    def _init():
        lo, hi = _segment_bounds(batch_ref[...])
        lo_acc[...] = lo
        hi_acc[...] = hi

    oh = _onehot(bi_ref[...], lo_acc[...], hi_acc[...])
    zc_g = _dot(oh, zc_ref[...], _DN_STD)
    rg = _dot(oh, rrep_ref[...], _DN_STD)
    hg = _dot(oh, hb_ref[...], _DN_STD)
    z = z_ref[...]
    zo_ref[...] = zc_g + (z - zc_g) * rg
    hn = h_ref[...] + hg
    mu = jnp.mean(hn, axis=1, keepdims=True)
    df = hn - mu
    v = jnp.mean(df * df, axis=1, keepdims=True)
    ho_ref[...] = df * lax.rsqrt(v + 1e-5) * gamma_ref[...] + beta_ref[...]


@jax.jit
def kernel(H, Z, block_id, batch_id, sigma, W1, b1, W2, b2, gamma, beta):
    z2 = Z.reshape(N, D_Z)
    bi_col = block_id.reshape(N, 1).astype(jnp.int32)
    batch_col = batch_id.reshape(N_BLOCKS, 1).astype(jnp.int32)
    sigma_r = sigma.reshape(1, N_CHANNEL)
    w1t = W1.T
    w2t = W2.T
    b1r = b1.reshape(1, D_HIDDEN)
    b2r = b2.reshape(1, D_HIDDEN)
    gamma_r = gamma.reshape(1, D_HIDDEN)
    beta_r = beta.reshape(1, D_HIDDEN)

    ngrid1 = N // _BLK1
    const = lambda shape: pl.BlockSpec(shape, lambda i: (0, 0))
    zc, rrep, h2, resc = pl.pallas_call(
        functools.partial(_stats_kernel, ngrid=ngrid1),
        grid=(ngrid1,),
        in_specs=[
            pl.BlockSpec((_BLK1, D_Z), lambda i: (i, 0)),
            pl.BlockSpec((_BLK1, 1), lambda i: (i, 0)),
            const((N_BLOCKS, 1)),
            const((1, N_CHANNEL)),
            const((N_CHANNEL * N_RBF, D_HIDDEN)),
            const((1, D_HIDDEN)),
            const((D_HIDDEN, D_HIDDEN)),
            const((1, D_HIDDEN)),
        ],
        out_specs=[
            const((N_BATCH, D_Z)),
            const((N_BATCH, D_Z)),
            const((N_BATCH, D_HIDDEN)),
            const((N_BATCH, N_CHANNEL)),
        ],
        out_shape=[
            jax.ShapeDtypeStruct((N_BATCH, D_Z), jnp.float32),
            jax.ShapeDtypeStruct((N_BATCH, D_Z), jnp.float32),
            jax.ShapeDtypeStruct((N_BATCH, D_HIDDEN), jnp.float32),
            jax.ShapeDtypeStruct((N_BATCH, N_CHANNEL), jnp.float32),
        ],
        scratch_shapes=[
            pltpu.VMEM((N_BATCH, D_Z), jnp.float32),
            pltpu.VMEM((N_BATCH, D_Z), jnp.float32),
            pltpu.VMEM((N_BATCH, 1), jnp.float32),
            pltpu.VMEM((1, N_BATCH), jnp.int32),
            pltpu.VMEM((1, N_BATCH), jnp.int32),
        ],
    )(z2, bi_col, batch_col, sigma_r, w1t, b1r, w2t, b2r)

    ngrid2 = N // _BLK2
    h_out, z_out = pl.pallas_call(
        _apply_kernel,
        grid=(ngrid2,),
        in_specs=[
            pl.BlockSpec((_BLK2, D_HIDDEN), lambda i: (i, 0)),
            pl.BlockSpec((_BLK2, D_Z), lambda i: (i, 0)),
            pl.BlockSpec((_BLK2, 1), lambda i: (i, 0)),
            const((N_BLOCKS, 1)),
            const((N_BATCH, D_Z)),
            const((N_BATCH, D_Z)),
            const((N_BATCH, D_HIDDEN)),
            const((1, D_HIDDEN)),
            const((1, D_HIDDEN)),
        ],
        out_specs=[
            pl.BlockSpec((_BLK2, D_HIDDEN), lambda i: (i, 0)),
            pl.BlockSpec((_BLK2, D_Z), lambda i: (i, 0)),
        ],
        out_shape=[
            jax.ShapeDtypeStruct((N, D_HIDDEN), jnp.float32),
            jax.ShapeDtypeStruct((N, D_Z), jnp.float32),
        ],
        scratch_shapes=[
            pltpu.VMEM((1, N_BATCH), jnp.int32),
            pltpu.VMEM((1, N_BATCH), jnp.int32),
        ],
    )(H, z2, bi_col, batch_col, zc, rrep, h2, gamma_r, beta_r)

    return (h_out, z_out.reshape(N, N_CHANNEL, 3),
            resc.reshape(N_BATCH, N_CHANNEL, 1))


# speedup vs baseline: 41.2958x; 1.4868x over previous
import functools

import jax
import jax.numpy as jnp
from jax import lax
from jax.experimental import pallas as pl
from jax.experimental.pallas import tpu as pltpu

N = 65536
N_BLOCKS = 4096
N_BATCH = 16
D_HIDDEN = 128
N_CHANNEL = 14
N_RBF = 16
CUTOFF = 7.0
D_Z = N_CHANNEL * 3

_BLK1 = 8192
_BLK2 = 4096

_DN_T0 = (((0,), (0,)), ((), ()))
_DN_STD = (((1,), (0,)), ((), ()))
_DN_T1 = (((1,), (1,)), ((), ()))


def _dot(a, b, dn):
    return lax.dot_general(a, b, dn, preferred_element_type=jnp.float32)


def _onehot(i, blk, us, usn):
    io0 = lax.broadcasted_iota(jnp.int32, (blk, N_BATCH), 0) + i * blk
    oh = jnp.logical_and(io0 >= us, io0 < usn)
    return jnp.where(oh, jnp.float32(1.0), jnp.float32(0.0))


def _silu(x):
    return x / (1.0 + jnp.exp(-x))


def _bounds_kernel(blk_ref, bat_ref, us_ref, usn_ref):
    blk = blk_ref[...]
    bat = bat_ref[...]
    starts = []
    for b in range(N_BATCH + 1):
        bs_b = jnp.sum((bat < b).astype(jnp.int32))
        starts.append(jnp.sum((blk < bs_b).astype(jnp.int32)))
    io1 = lax.broadcasted_iota(jnp.int32, (1, N_BATCH), 1)
    us = jnp.zeros((1, N_BATCH), jnp.int32)
    usn = jnp.zeros((1, N_BATCH), jnp.int32)
    for b in range(N_BATCH):
        us = jnp.where(io1 == b, starts[b], us)
        usn = jnp.where(io1 == b, starts[b + 1], usn)
    us_ref[...] = us
    usn_ref[...] = usn


def _stats_kernel(z_ref, us_ref, usn_ref, sigma_ref, w1t_ref, b1_ref,
                  w2t_ref, b2_ref,
                  zc_ref, rrep_ref, h2_ref, resc_ref,
                  s1_acc, s2_acc, *, ngrid):
    i = pl.program_id(0)

    @pl.when(i == 0)
    def _init():
        s1_acc[...] = jnp.zeros_like(s1_acc)
        s2_acc[...] = jnp.zeros_like(s2_acc)

    z = z_ref[...]
    oh = _onehot(i, _BLK1, us_ref[...], usn_ref[...])
    s1_acc[...] += _dot(oh, z, _DN_T0)
    s2_acc[...] += _dot(oh, z * z, _DN_T0)

    @pl.when(i == ngrid - 1)
    def _middle():
        s1 = s1_acc[...]
        s2 = s2_acc[...]
        cnt_row = (usn_ref[...] - us_ref[...]).astype(jnp.float32)
        r16 = lax.broadcasted_iota(jnp.int32, (N_BATCH, N_BATCH), 0)
        c16 = lax.broadcasted_iota(jnp.int32, (N_BATCH, N_BATCH), 1)
        eye = jnp.where(r16 == c16, jnp.float32(1.0), jnp.float32(0.0))
        cnt = _dot(eye, cnt_row, _DN_T1)
        cntc = jnp.maximum(cnt, 1.0)
        zc = s1 / cntc
        r_io = lax.broadcasted_iota(jnp.int32, (D_Z, N_CHANNEL), 0)
        c_io = lax.broadcasted_iota(jnp.int32, (D_Z, N_CHANNEL), 1)
        grp = jnp.where(jnp.logical_and(r_io >= 3 * c_io, r_io < 3 * c_io + 3),
                        jnp.float32(1.0), jnp.float32(0.0))
        t2 = _dot(s2 - s1 * s1 / cntc, grp, _DN_STD)
        denom = jnp.maximum(3.0 * cnt - 1.0, 1.0)
        std = jnp.sqrt(t2 / denom)
        resc = sigma_ref[...] / std
        resc_ref[...] = resc
        resc_safe = jnp.minimum(resc, jnp.float32(1e30))
        ce = lax.broadcasted_iota(jnp.int32, (N_CHANNEL, N_CHANNEL * N_RBF), 0)
        je = lax.broadcasted_iota(jnp.int32, (N_CHANNEL, N_CHANNEL * N_RBF), 1)
        exp_m = jnp.where(
            jnp.logical_and(je >= N_RBF * ce, je < N_RBF * ce + N_RBF),
            jnp.float32(1.0), jnp.float32(0.0))
        dsc = _dot(resc_safe / CUTOFF, exp_m, _DN_STD)
        p = 5.0
        ea = -(p + 1.0) * (p + 2.0) / 2.0
        eb = p * (p + 2.0)
        ec = -p * (p + 1.0) / 2.0
        d4 = (dsc * dsc) * (dsc * dsc)
        env = 1.0 / dsc + ea * d4 + eb * d4 * dsc + ec * d4 * dsc * dsc
        env = jnp.where(dsc < 1.0, env, 0.0)
        j_io = lax.broadcasted_iota(jnp.int32, (1, N_CHANNEL * N_RBF), 1)
        step = jnp.float32(1.0 / (N_RBF - 1))
        offs = (j_io % N_RBF).astype(jnp.float32) * step
        coeff = -0.5 / (step * step)
        gauss = jnp.exp(coeff * (dsc - offs) * (dsc - offs))
        rbf = env * gauss
        h1 = _silu(_dot(rbf, w1t_ref[...], _DN_STD) + b1_ref[...])
        h2 = _silu(_dot(h1, w2t_ref[...], _DN_STD) + b2_ref[...])
        h2_ref[...] = h2
        c3 = lax.broadcasted_iota(jnp.int32, (N_CHANNEL, D_Z), 0)
        j3 = lax.broadcasted_iota(jnp.int32, (N_CHANNEL, D_Z), 1)
        rep3 = jnp.where(jnp.logical_and(j3 >= 3 * c3, j3 < 3 * c3 + 3),
                         jnp.float32(1.0), jnp.float32(0.0))
        rrep_ref[...] = _dot(resc_safe, rep3, _DN_STD)
        zc_ref[...] = zc


def _apply_kernel(h_ref, z_ref, us_ref, usn_ref, zc_ref, rrep_ref,
                  hb_ref, gamma_ref, beta_ref,
                  ho_ref, zo_ref):
    i = pl.program_id(0)
    oh = _onehot(i, _BLK2, us_ref[...], usn_ref[...])
    zc_g = _dot(oh, zc_ref[...], _DN_STD)
    rg = _dot(oh, rrep_ref[...], _DN_STD)
    hg = _dot(oh, hb_ref[...], _DN_STD)
    z = z_ref[...]
    zo_ref[...] = zc_g + (z - zc_g) * rg
    hn = h_ref[...] + hg
    mu = jnp.mean(hn, axis=1, keepdims=True)
    df = hn - mu
    v = jnp.mean(df * df, axis=1, keepdims=True)
    ho_ref[...] = df * lax.rsqrt(v + 1e-5) * gamma_ref[...] + beta_ref[...]


@jax.jit
def kernel(H, Z, block_id, batch_id, sigma, W1, b1, W2, b2, gamma, beta):
    z2 = Z.reshape(N, D_Z)
    blk_nat = block_id.reshape(N // 128, 128).astype(jnp.int32)
    bat_nat = batch_id.reshape(N_BLOCKS // 128, 128).astype(jnp.int32)
    sigma_r = sigma.reshape(1, N_CHANNEL)
    w1t = W1.T
    w2t = W2.T
    b1r = b1.reshape(1, D_HIDDEN)
    b2r = b2.reshape(1, D_HIDDEN)
    gamma_r = gamma.reshape(1, D_HIDDEN)
    beta_r = beta.reshape(1, D_HIDDEN)

    full = lambda shape: pl.BlockSpec(shape, lambda i: (0, 0))
    us, usn = pl.pallas_call(
        _bounds_kernel,
        in_specs=[
            pl.BlockSpec((N // 128, 128), lambda: (0, 0)),
            pl.BlockSpec((N_BLOCKS // 128, 128), lambda: (0, 0)),
        ],
        out_specs=[
            pl.BlockSpec((1, N_BATCH), lambda: (0, 0)),
            pl.BlockSpec((1, N_BATCH), lambda: (0, 0)),
        ],
        out_shape=[
            jax.ShapeDtypeStruct((1, N_BATCH), jnp.int32),
            jax.ShapeDtypeStruct((1, N_BATCH), jnp.int32),
        ],
    )(blk_nat, bat_nat)

    ngrid1 = N // _BLK1
    zc, rrep, h2, resc = pl.pallas_call(
        functools.partial(_stats_kernel, ngrid=ngrid1),
        grid=(ngrid1,),
        in_specs=[
            pl.BlockSpec((_BLK1, D_Z), lambda i: (i, 0)),
            full((1, N_BATCH)),
            full((1, N_BATCH)),
            full((1, N_CHANNEL)),
            full((N_CHANNEL * N_RBF, D_HIDDEN)),
            full((1, D_HIDDEN)),
            full((D_HIDDEN, D_HIDDEN)),
            full((1, D_HIDDEN)),
        ],
        out_specs=[
            full((N_BATCH, D_Z)),
            full((N_BATCH, D_Z)),
            full((N_BATCH, D_HIDDEN)),
            full((N_BATCH, N_CHANNEL)),
        ],
        out_shape=[
            jax.ShapeDtypeStruct((N_BATCH, D_Z), jnp.float32),
            jax.ShapeDtypeStruct((N_BATCH, D_Z), jnp.float32),
            jax.ShapeDtypeStruct((N_BATCH, D_HIDDEN), jnp.float32),
            jax.ShapeDtypeStruct((N_BATCH, N_CHANNEL), jnp.float32),
        ],
        scratch_shapes=[
            pltpu.VMEM((N_BATCH, D_Z), jnp.float32),
            pltpu.VMEM((N_BATCH, D_Z), jnp.float32),
        ],
    )(z2, us, usn, sigma_r, w1t, b1r, w2t, b2r)

    ngrid2 = N // _BLK2
    h_out, z_out = pl.pallas_call(
        _apply_kernel,
        grid=(ngrid2,),
        in_specs=[
            pl.BlockSpec((_BLK2, D_HIDDEN), lambda i: (i, 0)),
            pl.BlockSpec((_BLK2, D_Z), lambda i: (i, 0)),
            full((1, N_BATCH)),
            full((1, N_BATCH)),
            full((N_BATCH, D_Z)),
            full((N_BATCH, D_Z)),
            full((N_BATCH, D_HIDDEN)),
            full((1, D_HIDDEN)),
            full((1, D_HIDDEN)),
        ],
        out_specs=[
            pl.BlockSpec((_BLK2, D_HIDDEN), lambda i: (i, 0)),
            pl.BlockSpec((_BLK2, D_Z), lambda i: (i, 0)),
        ],
        out_shape=[
            jax.ShapeDtypeStruct((N, D_HIDDEN), jnp.float32),
            jax.ShapeDtypeStruct((N, D_Z), jnp.float32),
        ],
    )(H, z2, us, usn, zc, rrep, h2, gamma_r, beta_r)

    return (h_out, z_out.reshape(N, N_CHANNEL, 3),
            resc.reshape(N_BATCH, N_CHANNEL, 1))


# speedup vs baseline: 111.0935x; 2.6902x over previous
import functools

import jax
import jax.numpy as jnp
from jax import lax
from jax.experimental import pallas as pl
from jax.experimental.pallas import tpu as pltpu

N = 65536
N_BLOCKS = 4096
N_BATCH = 16
D_HIDDEN = 128
N_CHANNEL = 14
N_RBF = 16
CUTOFF = 7.0

_BLKN1 = 8192
_BLKN2 = 8192
_BLKH = 4096

_DN_STD = (((1,), (0,)), ((), ()))
_DN_T1 = (((1,), (1,)), ((), ()))


def _dot(a, b, dn=_DN_STD):
    return lax.dot_general(a, b, dn, preferred_element_type=jnp.float32)


def _silu(x):
    return x / (1.0 + jnp.exp(-x))


def _bounds_kernel(blk_ref, bat_ref, usr_ref, usnr_ref, usc_ref, usnc_ref):
    blk = blk_ref[...]
    bat = bat_ref[...]
    starts = []
    for b in range(N_BATCH + 1):
        bs_b = jnp.sum((bat < b).astype(jnp.int32))
        starts.append(jnp.sum((blk < bs_b).astype(jnp.int32)))
    io_r = lax.broadcasted_iota(jnp.int32, (1, N_BATCH), 1)
    io_c = lax.broadcasted_iota(jnp.int32, (N_BATCH, 1), 0)
    usr = jnp.zeros((1, N_BATCH), jnp.int32)
    usnr = jnp.zeros((1, N_BATCH), jnp.int32)
    usc = jnp.zeros((N_BATCH, 1), jnp.int32)
    usnc = jnp.zeros((N_BATCH, 1), jnp.int32)
    for b in range(N_BATCH):
        usr = jnp.where(io_r == b, starts[b], usr)
        usnr = jnp.where(io_r == b, starts[b + 1], usnr)
        usc = jnp.where(io_c == b, starts[b], usc)
        usnc = jnp.where(io_c == b, starts[b + 1], usnc)
    usr_ref[...] = usr
    usnr_ref[...] = usnr
    usc_ref[...] = usc
    usnc_ref[...] = usnc


def _eye16():
    r16 = lax.broadcasted_iota(jnp.int32, (N_BATCH, N_BATCH), 0)
    c16 = lax.broadcasted_iota(jnp.int32, (N_BATCH, N_BATCH), 1)
    return jnp.where(r16 == c16, jnp.float32(1.0), jnp.float32(0.0))


def _stats_kernel(zt_ref, usr_ref, usnr_ref, sigma_ref, w1_ref, b1_ref,
                  w2_ref, b2_ref,
                  ztab_ref, h2_ref, resc_ref,
                  s1_acc, s2_acc, *, ngrid):
    i = pl.program_id(0)

    @pl.when(i == 0)
    def _init():
        s1_acc[...] = jnp.zeros_like(s1_acc)
        s2_acc[...] = jnp.zeros_like(s2_acc)

    io0 = lax.broadcasted_iota(jnp.int32, (_BLKN1, N_BATCH), 0) + i * _BLKN1
    oh = jnp.where(
        jnp.logical_and(io0 >= usr_ref[...], io0 < usnr_ref[...]),
        jnp.float32(1.0), jnp.float32(0.0))
    z3 = zt_ref[...]
    for a in range(3):
        za = z3[a]
        s1_acc[a, :, :] += _dot(za, oh)
        s2_acc[a, :, :] += _dot(za * za, oh)

    @pl.when(i == ngrid - 1)
    def _middle():
        s1a = s1_acc[...]
        s2a = s2_acc[...]
        cnt = (usnr_ref[...] - usr_ref[...]).astype(jnp.float32)
        cntc = jnp.maximum(cnt, 1.0)
        zc3 = s1a / cntc
        e = s2a - s1a * s1a / cntc
        t2 = e[0] + e[1] + e[2]
        denom = jnp.maximum(3.0 * cnt - 1.0, 1.0)
        std = jnp.sqrt(t2 / denom)
        resc_t = sigma_ref[...] / std
        resc_ref[...] = _dot(_eye16(), resc_t, _DN_T1)
        resc_safe = jnp.minimum(resc_t, jnp.float32(1e30))
        for a in range(3):
            ztab_ref[a, :, :] = zc3[a]
        ztab_ref[3, :, :] = resc_safe
        nf = N_CHANNEL * N_RBF
        ce = lax.broadcasted_iota(jnp.int32, (nf, N_CHANNEL), 0)
        je = lax.broadcasted_iota(jnp.int32, (nf, N_CHANNEL), 1)
        exp_m = jnp.where(
            jnp.logical_and(ce >= N_RBF * je, ce < N_RBF * je + N_RBF),
            jnp.float32(1.0), jnp.float32(0.0))
        dsc = _dot(exp_m, resc_safe / CUTOFF)
        p = 5.0
        ea = -(p + 1.0) * (p + 2.0) / 2.0
        eb = p * (p + 2.0)
        ec = -p * (p + 1.0) / 2.0
        d4 = (dsc * dsc) * (dsc * dsc)
        env = 1.0 / dsc + ea * d4 + eb * d4 * dsc + ec * d4 * dsc * dsc
        env = jnp.where(dsc < 1.0, env, 0.0)
        f_io = lax.broadcasted_iota(jnp.int32, (nf, 1), 0)
        step = jnp.float32(1.0 / (N_RBF - 1))
        offs = (f_io % N_RBF).astype(jnp.float32) * step
        coeff = -0.5 / (step * step)
        gauss = jnp.exp(coeff * (dsc - offs) * (dsc - offs))
        rbf = env * gauss
        h1 = _silu(_dot(w1_ref[...], rbf) + b1_ref[...])
        h2 = _silu(_dot(w2_ref[...], h1) + b2_ref[...])
        h2_ref[...] = _dot(_eye16(), h2, _DN_T1)


def _zapply_kernel(zt_ref, usc_ref, usnc_ref, ztab_ref, zo_ref):
    i = pl.program_id(0)
    io1 = lax.broadcasted_iota(jnp.int32, (N_BATCH, _BLKN2), 1) + i * _BLKN2
    oh = jnp.where(
        jnp.logical_and(io1 >= usc_ref[...], io1 < usnc_ref[...]),
        jnp.float32(1.0), jnp.float32(0.0))
    rg = _dot(ztab_ref[3], oh)
    z3 = zt_ref[...]
    for a in range(3):
        zcg = _dot(ztab_ref[a], oh)
        za = z3[a]
        zo_ref[a, :, :] = zcg + (za - zcg) * rg


def _happly_kernel(h_ref, usr_ref, usnr_ref, h2_ref, gamma_ref, beta_ref,
                   ho_ref):
    i = pl.program_id(0)
    io0 = lax.broadcasted_iota(jnp.int32, (_BLKH, N_BATCH), 0) + i * _BLKH
    oh = jnp.where(
        jnp.logical_and(io0 >= usr_ref[...], io0 < usnr_ref[...]),
        jnp.float32(1.0), jnp.float32(0.0))
    hg = _dot(oh, h2_ref[...])
    hn = h_ref[...] + hg
    mu = jnp.mean(hn, axis=1, keepdims=True)
    df = hn - mu
    v = jnp.mean(df * df, axis=1, keepdims=True)
    ho_ref[...] = df * lax.rsqrt(v + 1e-5) * gamma_ref[...] + beta_ref[...]


@jax.jit
def kernel(H, Z, block_id, batch_id, sigma, W1, b1, W2, b2, gamma, beta):
    zt = jnp.transpose(Z, (2, 1, 0))
    blk_nat = block_id.reshape(N // 128, 128).astype(jnp.int32)
    bat_nat = batch_id.reshape(N_BLOCKS // 128, 128).astype(jnp.int32)
    sigma_c = sigma.reshape(N_CHANNEL, 1)
    b1c = b1.reshape(D_HIDDEN, 1)
    b2c = b2.reshape(D_HIDDEN, 1)
    gamma_r = gamma.reshape(1, D_HIDDEN)
    beta_r = beta.reshape(1, D_HIDDEN)

    usr, usnr, usc, usnc = pl.pallas_call(
        _bounds_kernel,
        in_specs=[
            pl.BlockSpec((N // 128, 128), lambda: (0, 0)),
            pl.BlockSpec((N_BLOCKS // 128, 128), lambda: (0, 0)),
        ],
        out_specs=[
            pl.BlockSpec((1, N_BATCH), lambda: (0, 0)),
            pl.BlockSpec((1, N_BATCH), lambda: (0, 0)),
            pl.BlockSpec((N_BATCH, 1), lambda: (0, 0)),
            pl.BlockSpec((N_BATCH, 1), lambda: (0, 0)),
        ],
        out_shape=[
            jax.ShapeDtypeStruct((1, N_BATCH), jnp.int32),
            jax.ShapeDtypeStruct((1, N_BATCH), jnp.int32),
            jax.ShapeDtypeStruct((N_BATCH, 1), jnp.int32),
            jax.ShapeDtypeStruct((N_BATCH, 1), jnp.int32),
        ],
    )(blk_nat, bat_nat)

    full = lambda shape: pl.BlockSpec(shape, lambda i: tuple(0 for _ in shape))
    ngrid1 = N // _BLKN1
    ztab, h2, resc = pl.pallas_call(
        functools.partial(_stats_kernel, ngrid=ngrid1),
        grid=(ngrid1,),
        in_specs=[
            pl.BlockSpec((3, N_CHANNEL, _BLKN1), lambda i: (0, 0, i)),
            full((1, N_BATCH)),
            full((1, N_BATCH)),
            full((N_CHANNEL, 1)),
            full((D_HIDDEN, N_CHANNEL * N_RBF)),
            full((D_HIDDEN, 1)),
            full((D_HIDDEN, D_HIDDEN)),
            full((D_HIDDEN, 1)),
        ],
        out_specs=[
            full((4, N_CHANNEL, N_BATCH)),
            full((N_BATCH, D_HIDDEN)),
            full((N_BATCH, N_CHANNEL)),
        ],
        out_shape=[
            jax.ShapeDtypeStruct((4, N_CHANNEL, N_BATCH), jnp.float32),
            jax.ShapeDtypeStruct((N_BATCH, D_HIDDEN), jnp.float32),
            jax.ShapeDtypeStruct((N_BATCH, N_CHANNEL), jnp.float32),
        ],
        scratch_shapes=[
            pltpu.VMEM((3, N_CHANNEL, N_BATCH), jnp.float32),
            pltpu.VMEM((3, N_CHANNEL, N_BATCH), jnp.float32),
        ],
    )(zt, usr, usnr, sigma_c, W1, b1c, W2, b2c)

    ngrid2 = N // _BLKN2
    zo_t = pl.pallas_call(
        _zapply_kernel,
        grid=(ngrid2,),
        in_specs=[
            pl.BlockSpec((3, N_CHANNEL, _BLKN2), lambda i: (0, 0, i)),
            full((N_BATCH, 1)),
            full((N_BATCH, 1)),
            full((4, N_CHANNEL, N_BATCH)),
        ],
        out_specs=pl.BlockSpec((3, N_CHANNEL, _BLKN2), lambda i: (0, 0, i)),
        out_shape=jax.ShapeDtypeStruct((3, N_CHANNEL, N), jnp.float32),
    )(zt, usc, usnc, ztab)

    ngrid3 = N // _BLKH
    h_out = pl.pallas_call(
        _happly_kernel,
        grid=(ngrid3,),
        in_specs=[
            pl.BlockSpec((_BLKH, D_HIDDEN), lambda i: (i, 0)),
            full((1, N_BATCH)),
            full((1, N_BATCH)),
            full((N_BATCH, D_HIDDEN)),
            full((1, D_HIDDEN)),
            full((1, D_HIDDEN)),
        ],
        out_specs=pl.BlockSpec((_BLKH, D_HIDDEN), lambda i: (i, 0)),
        out_shape=jax.ShapeDtypeStruct((N, D_HIDDEN), jnp.float32),
    )(H, usr, usnr, h2, gamma_r, beta_r)

    return (h_out, jnp.transpose(zo_t, (2, 1, 0)),
            resc.reshape(N_BATCH, N_CHANNEL, 1))


# speedup vs baseline: 142.6482x; 1.2840x over previous
import functools

import jax
import jax.numpy as jnp
from jax import lax
from jax.experimental import pallas as pl
from jax.experimental.pallas import tpu as pltpu

N = 65536
N_BLOCKS = 4096
N_BATCH = 16
D_HIDDEN = 128
N_CHANNEL = 14
N_RBF = 16
CUTOFF = 7.0

_BLKN1 = 16384
_BLKN2 = 16384
_BLKH = 8192

_DN_STD = (((1,), (0,)), ((), ()))
_DN_T0 = (((0,), (0,)), ((), ()))
_DN_T1 = (((1,), (1,)), ((), ()))


def _dot(a, b, dn=_DN_STD):
    return lax.dot_general(a, b, dn, preferred_element_type=jnp.float32)


def _silu(x):
    return x / (1.0 + jnp.exp(-x))


def _eye16():
    r16 = lax.broadcasted_iota(jnp.int32, (N_BATCH, N_BATCH), 0)
    c16 = lax.broadcasted_iota(jnp.int32, (N_BATCH, N_BATCH), 1)
    return jnp.where(r16 == c16, jnp.float32(1.0), jnp.float32(0.0))


def _onehot_t(i, blkn, usc, usnc):
    io1 = lax.broadcasted_iota(jnp.int32, (N_BATCH, blkn), 1) + i * blkn
    return jnp.where(jnp.logical_and(io1 >= usc, io1 < usnc),
                     jnp.float32(1.0), jnp.float32(0.0))


def _stats_kernel(zt_ref, blk_ref, bat_ref, sigma_ref, w1_ref, b1_ref,
                  w2_ref, b2_ref,
                  ztab_ref, h2_ref, resc_ref, usr_ref, usnr_ref,
                  usc_ref, usnc_ref,
                  s1_acc, s2_acc, usc_s, usnc_s, *, ngrid):
    i = pl.program_id(0)

    @pl.when(i == 0)
    def _init():
        blk = blk_ref[...]
        bat = bat_ref[...]
        starts = []
        for b in range(N_BATCH + 1):
            bs_b = jnp.sum((bat < b).astype(jnp.int32))
            starts.append(jnp.sum((blk < bs_b).astype(jnp.int32)))
        io_r = lax.broadcasted_iota(jnp.int32, (1, N_BATCH), 1)
        io_c = lax.broadcasted_iota(jnp.int32, (N_BATCH, 1), 0)
        usr = jnp.zeros((1, N_BATCH), jnp.int32)
        usnr = jnp.zeros((1, N_BATCH), jnp.int32)
        usc = jnp.zeros((N_BATCH, 1), jnp.int32)
        usnc = jnp.zeros((N_BATCH, 1), jnp.int32)
        for b in range(N_BATCH):
            usr = jnp.where(io_r == b, starts[b], usr)
            usnr = jnp.where(io_r == b, starts[b + 1], usnr)
            usc = jnp.where(io_c == b, starts[b], usc)
            usnc = jnp.where(io_c == b, starts[b + 1], usnc)
        usr_ref[...] = usr
        usnr_ref[...] = usnr
        usc_ref[...] = usc
        usnc_ref[...] = usnc
        usc_s[...] = usc
        usnc_s[...] = usnc
        s1_acc[...] = jnp.zeros_like(s1_acc)
        s2_acc[...] = jnp.zeros_like(s2_acc)

    oh = _onehot_t(i, _BLKN1, usc_s[...], usnc_s[...])
    z3 = zt_ref[...]
    for a in range(3):
        za = z3[a]
        s1_acc[a, :, :] += _dot(za, oh, _DN_T1)
        s2_acc[a, :, :] += _dot(za * za, oh, _DN_T1)

    @pl.when(i == ngrid - 1)
    def _middle():
        s1a = s1_acc[...]
        s2a = s2_acc[...]
        cnt = (usnc_s[...] - usc_s[...]).astype(jnp.float32)
        cnt_r = _dot(cnt, _eye16(), _DN_T0)
        cntc = jnp.maximum(cnt_r, 1.0)
        zc3 = s1a / cntc
        e = s2a - s1a * s1a / cntc
        t2 = e[0] + e[1] + e[2]
        denom = jnp.maximum(3.0 * cnt_r - 1.0, 1.0)
        std = jnp.sqrt(t2 / denom)
        resc_t = sigma_ref[...] / std
        resc_ref[...] = _dot(_eye16(), resc_t, _DN_T1)
        resc_safe = jnp.minimum(resc_t, jnp.float32(1e30))
        for a in range(3):
            ztab_ref[a, :, :] = zc3[a]
        ztab_ref[3, :, :] = resc_safe
        nf = N_CHANNEL * N_RBF
        ce = lax.broadcasted_iota(jnp.int32, (nf, N_CHANNEL), 0)
        je = lax.broadcasted_iota(jnp.int32, (nf, N_CHANNEL), 1)
        exp_m = jnp.where(
            jnp.logical_and(ce >= N_RBF * je, ce < N_RBF * je + N_RBF),
            jnp.float32(1.0), jnp.float32(0.0))
        dsc = _dot(exp_m, resc_safe / CUTOFF)
        p = 5.0
        ea = -(p + 1.0) * (p + 2.0) / 2.0
        eb = p * (p + 2.0)
        ec = -p * (p + 1.0) / 2.0
        d4 = (dsc * dsc) * (dsc * dsc)
        env = 1.0 / dsc + ea * d4 + eb * d4 * dsc + ec * d4 * dsc * dsc
        env = jnp.where(dsc < 1.0, env, 0.0)
        f_io = lax.broadcasted_iota(jnp.int32, (nf, 1), 0)
        step = jnp.float32(1.0 / (N_RBF - 1))
        offs = (f_io % N_RBF).astype(jnp.float32) * step
        coeff = -0.5 / (step * step)
        gauss = jnp.exp(coeff * (dsc - offs) * (dsc - offs))
        rbf = env * gauss
        h1 = _silu(_dot(w1_ref[...], rbf) + b1_ref[...])
        h2 = _silu(_dot(w2_ref[...], h1) + b2_ref[...])
        h2_ref[...] = _dot(_eye16(), h2, _DN_T1)


def _zapply_kernel(zt_ref, usc_ref, usnc_ref, ztab_ref, zo_ref):
    i = pl.program_id(0)
    oh = _onehot_t(i, _BLKN2, usc_ref[...], usnc_ref[...])
    rg = _dot(ztab_ref[3], oh)
    z3 = zt_ref[...]
    for a in range(3):
        zcg = _dot(ztab_ref[a], oh)
        za = z3[a]
        zo_ref[a, :, :] = zcg + (za - zcg) * rg


def _happly_kernel(h_ref, usc_ref, usnc_ref, h2_ref, gamma_ref, beta_ref,
                   ho_ref):
    i = pl.program_id(0)
    oh = _onehot_t(i, _BLKH, usc_ref[...], usnc_ref[...])
    hg = _dot(oh, h2_ref[...], _DN_T0)
    hn = h_ref[...] + hg
    mu = jnp.mean(hn, axis=1, keepdims=True)
    df = hn - mu
    v = jnp.mean(df * df, axis=1, keepdims=True)
    ho_ref[...] = df * lax.rsqrt(v + 1e-5) * gamma_ref[...] + beta_ref[...]


@jax.jit
def kernel(H, Z, block_id, batch_id, sigma, W1, b1, W2, b2, gamma, beta):
    zt = jnp.transpose(Z, (2, 1, 0))
    blk_nat = block_id.reshape(N // 128, 128).astype(jnp.int32)
    bat_nat = batch_id.reshape(N_BLOCKS // 128, 128).astype(jnp.int32)
    sigma_c = sigma.reshape(N_CHANNEL, 1)
    b1c = b1.reshape(D_HIDDEN, 1)
    b2c = b2.reshape(D_HIDDEN, 1)
    gamma_r = gamma.reshape(1, D_HIDDEN)
    beta_r = beta.reshape(1, D_HIDDEN)

    full = lambda shape: pl.BlockSpec(shape, lambda i: tuple(0 for _ in shape))
    ngrid1 = N // _BLKN1
    ztab, h2, resc, usr, usnr, usc, usnc = pl.pallas_call(
        functools.partial(_stats_kernel, ngrid=ngrid1),
        grid=(ngrid1,),
        in_specs=[
            pl.BlockSpec((3, N_CHANNEL, _BLKN1), lambda i: (0, 0, i)),
            full((N // 128, 128)),
            full((N_BLOCKS // 128, 128)),
            full((N_CHANNEL, 1)),
            full((D_HIDDEN, N_CHANNEL * N_RBF)),
            full((D_HIDDEN, 1)),
            full((D_HIDDEN, D_HIDDEN)),
            full((D_HIDDEN, 1)),
        ],
        out_specs=[
            full((4, N_CHANNEL, N_BATCH)),
            full((N_BATCH, D_HIDDEN)),
            full((N_BATCH, N_CHANNEL)),
            full((1, N_BATCH)),
            full((1, N_BATCH)),
            full((N_BATCH, 1)),
            full((N_BATCH, 1)),
        ],
        out_shape=[
            jax.ShapeDtypeStruct((4, N_CHANNEL, N_BATCH), jnp.float32),
            jax.ShapeDtypeStruct((N_BATCH, D_HIDDEN), jnp.float32),
            jax.ShapeDtypeStruct((N_BATCH, N_CHANNEL), jnp.float32),
            jax.ShapeDtypeStruct((1, N_BATCH), jnp.int32),
            jax.ShapeDtypeStruct((1, N_BATCH), jnp.int32),
            jax.ShapeDtypeStruct((N_BATCH, 1), jnp.int32),
            jax.ShapeDtypeStruct((N_BATCH, 1), jnp.int32),
        ],
        scratch_shapes=[
            pltpu.VMEM((3, N_CHANNEL, N_BATCH), jnp.float32),
            pltpu.VMEM((3, N_CHANNEL, N_BATCH), jnp.float32),
            pltpu.VMEM((N_BATCH, 1), jnp.int32),
            pltpu.VMEM((N_BATCH, 1), jnp.int32),
        ],
    )(zt, blk_nat, bat_nat, sigma_c, W1, b1c, W2, b2c)

    ngrid2 = N // _BLKN2
    zo_t = pl.pallas_call(
        _zapply_kernel,
        grid=(ngrid2,),
        in_specs=[
            pl.BlockSpec((3, N_CHANNEL, _BLKN2), lambda i: (0, 0, i)),
            full((N_BATCH, 1)),
            full((N_BATCH, 1)),
            full((4, N_CHANNEL, N_BATCH)),
        ],
        out_specs=pl.BlockSpec((3, N_CHANNEL, _BLKN2), lambda i: (0, 0, i)),
        out_shape=jax.ShapeDtypeStruct((3, N_CHANNEL, N), jnp.float32),
    )(zt, usc, usnc, ztab)

    ngrid3 = N // _BLKH
    h_out = pl.pallas_call(
        _happly_kernel,
        grid=(ngrid3,),
        in_specs=[
            pl.BlockSpec((_BLKH, D_HIDDEN), lambda i: (i, 0)),
            full((N_BATCH, 1)),
            full((N_BATCH, 1)),
            full((N_BATCH, D_HIDDEN)),
            full((1, D_HIDDEN)),
            full((1, D_HIDDEN)),
        ],
        out_specs=pl.BlockSpec((_BLKH, D_HIDDEN), lambda i: (i, 0)),
        out_shape=jax.ShapeDtypeStruct((N, D_HIDDEN), jnp.float32),
    )(H, usc, usnc, h2, gamma_r, beta_r)

    return (h_out, jnp.transpose(zo_t, (2, 1, 0)),
            resc.reshape(N_BATCH, N_CHANNEL, 1))


# speedup vs baseline: 156.7833x; 1.0991x over previous
import functools

import jax
import jax.numpy as jnp
from jax import lax
from jax.experimental import pallas as pl
from jax.experimental.pallas import tpu as pltpu

N = 65536
N_BLOCKS = 4096
N_BATCH = 16
D_HIDDEN = 128
N_CHANNEL = 14
N_RBF = 16
CUTOFF = 7.0

_BLKN = 16384
_BLKH = 8192

_DN_STD = (((1,), (0,)), ((), ()))
_DN_T0 = (((0,), (0,)), ((), ()))
_DN_T1 = (((1,), (1,)), ((), ()))


def _dot(a, b, dn=_DN_STD):
    return lax.dot_general(a, b, dn, preferred_element_type=jnp.float32)


def _silu(x):
    return x / (1.0 + jnp.exp(-x))


def _eye16():
    r16 = lax.broadcasted_iota(jnp.int32, (N_BATCH, N_BATCH), 0)
    c16 = lax.broadcasted_iota(jnp.int32, (N_BATCH, N_BATCH), 1)
    return jnp.where(r16 == c16, jnp.float32(1.0), jnp.float32(0.0))


def _onehot_t(base, blkn, usc, usnc):
    io1 = lax.broadcasted_iota(jnp.int32, (N_BATCH, blkn), 1) + base
    return jnp.where(jnp.logical_and(io1 >= usc, io1 < usnc),
                     jnp.float32(1.0), jnp.float32(0.0))


def _zpass_kernel(zt_ref, blk_ref, bat_ref, sigma_ref, w1_ref, b1_ref,
                  w2_ref, b2_ref,
                  zo_ref, h2_ref, resc_ref, usc_ref, usnc_ref,
                  s1_acc, s2_acc, usc_s, usnc_s, ztab_s, zcache, *, ng):
    i = pl.program_id(0)

    @pl.when(i == 0)
    def _init():
        blk = blk_ref[...]
        bat = bat_ref[...]
        starts = []
        for b in range(N_BATCH + 1):
            bs_b = jnp.sum((bat < b).astype(jnp.int32))
            starts.append(jnp.sum((blk < bs_b).astype(jnp.int32)))
        io_c = lax.broadcasted_iota(jnp.int32, (N_BATCH, 1), 0)
        usc = jnp.zeros((N_BATCH, 1), jnp.int32)
        usnc = jnp.zeros((N_BATCH, 1), jnp.int32)
        for b in range(N_BATCH):
            usc = jnp.where(io_c == b, starts[b], usc)
            usnc = jnp.where(io_c == b, starts[b + 1], usnc)
        usc_ref[...] = usc
        usnc_ref[...] = usnc
        usc_s[...] = usc
        usnc_s[...] = usnc
        s1_acc[...] = jnp.zeros_like(s1_acc)
        s2_acc[...] = jnp.zeros_like(s2_acc)
        ztab_s[...] = jnp.zeros_like(ztab_s)

    @pl.when(i < ng)
    def _stats():
        oh = _onehot_t(i * _BLKN, _BLKN, usc_s[...], usnc_s[...])
        z3 = zt_ref[...]
        zcache[:, :, pl.ds(i * _BLKN, _BLKN)] = z3
        for a in range(3):
            za = z3[a]
            s1_acc[a, :, :] += _dot(za, oh, _DN_T1)
            s2_acc[a, :, :] += _dot(za * za, oh, _DN_T1)

    @pl.when(i == ng - 1)
    def _middle():
        s1a = s1_acc[...]
        s2a = s2_acc[...]
        cnt = (usnc_s[...] - usc_s[...]).astype(jnp.float32)
        cnt_r = _dot(cnt, _eye16(), _DN_T0)
        cntc = jnp.maximum(cnt_r, 1.0)
        zc3 = s1a / cntc
        e = s2a - s1a * s1a / cntc
        t2 = e[0] + e[1] + e[2]
        denom = jnp.maximum(3.0 * cnt_r - 1.0, 1.0)
        std = jnp.sqrt(t2 / denom)
        resc_t = sigma_ref[...] / std
        resc_ref[...] = _dot(_eye16(), resc_t, _DN_T1)
        resc_safe = jnp.minimum(resc_t, jnp.float32(1e30))
        for a in range(3):
            ztab_s[16 * a:16 * a + N_CHANNEL, :] = zc3[a]
        ztab_s[48:48 + N_CHANNEL, :] = resc_safe
        nf = N_CHANNEL * N_RBF
        ce = lax.broadcasted_iota(jnp.int32, (nf, N_CHANNEL), 0)
        je = lax.broadcasted_iota(jnp.int32, (nf, N_CHANNEL), 1)
        exp_m = jnp.where(
            jnp.logical_and(ce >= N_RBF * je, ce < N_RBF * je + N_RBF),
            jnp.float32(1.0), jnp.float32(0.0))
        dsc = _dot(exp_m, resc_safe / CUTOFF)
        p = 5.0
        ea = -(p + 1.0) * (p + 2.0) / 2.0
        eb = p * (p + 2.0)
        ec = -p * (p + 1.0) / 2.0
        d4 = (dsc * dsc) * (dsc * dsc)
        env = 1.0 / dsc + ea * d4 + eb * d4 * dsc + ec * d4 * dsc * dsc
        env = jnp.where(dsc < 1.0, env, 0.0)
        f_io = lax.broadcasted_iota(jnp.int32, (nf, 1), 0)
        step = jnp.float32(1.0 / (N_RBF - 1))
        offs = (f_io % N_RBF).astype(jnp.float32) * step
        coeff = -0.5 / (step * step)
        gauss = jnp.exp(coeff * (dsc - offs) * (dsc - offs))
        rbf = env * gauss
        h1 = _silu(_dot(w1_ref[...], rbf) + b1_ref[...])
        h2 = _silu(_dot(w2_ref[...], h1) + b2_ref[...])
        h2_ref[...] = _dot(_eye16(), h2, _DN_T1)

    @pl.when(i >= ng)
    def _zapply():
        j = i - ng
        oh = _onehot_t(j * _BLKN, _BLKN, usc_s[...], usnc_s[...])
        g = _dot(ztab_s[...], oh)
        rg = g[48:48 + N_CHANNEL, :]
        z3 = zcache[:, :, pl.ds(j * _BLKN, _BLKN)]
        for a in range(3):
            zcg = g[16 * a:16 * a + N_CHANNEL, :]
            zo_ref[a, :, :] = zcg + (z3[a] - zcg) * rg


def _happly_kernel(h_ref, usc_ref, usnc_ref, h2_ref, gamma_ref, beta_ref,
                   ho_ref):
    i = pl.program_id(0)
    oh = _onehot_t(i * _BLKH, _BLKH, usc_ref[...], usnc_ref[...])
    hg = _dot(oh, h2_ref[...], _DN_T0)
    hn = h_ref[...] + hg
    mu = jnp.mean(hn, axis=1, keepdims=True)
    df = hn - mu
    v = jnp.mean(df * df, axis=1, keepdims=True)
    ho_ref[...] = df * lax.rsqrt(v + 1e-5) * gamma_ref[...] + beta_ref[...]


@jax.jit
def kernel(H, Z, block_id, batch_id, sigma, W1, b1, W2, b2, gamma, beta):
    zt = jnp.transpose(Z, (2, 1, 0))
    blk_nat = block_id.reshape(N // 128, 128).astype(jnp.int32)
    bat_nat = batch_id.reshape(N_BLOCKS // 128, 128).astype(jnp.int32)
    sigma_c = sigma.reshape(N_CHANNEL, 1)
    b1c = b1.reshape(D_HIDDEN, 1)
    b2c = b2.reshape(D_HIDDEN, 1)
    gamma_r = gamma.reshape(1, D_HIDDEN)
    beta_r = beta.reshape(1, D_HIDDEN)

    full = lambda shape: pl.BlockSpec(shape, lambda i: tuple(0 for _ in shape))
    ng = N // _BLKN
    zo_t, h2, resc, usc, usnc = pl.pallas_call(
        functools.partial(_zpass_kernel, ng=ng),
        grid=(2 * ng,),
        in_specs=[
            pl.BlockSpec((3, N_CHANNEL, _BLKN),
                         lambda i: (0, 0, jnp.minimum(i, N // _BLKN - 1))),
            full((N // 128, 128)),
            full((N_BLOCKS // 128, 128)),
            full((N_CHANNEL, 1)),
            full((D_HIDDEN, N_CHANNEL * N_RBF)),
            full((D_HIDDEN, 1)),
            full((D_HIDDEN, D_HIDDEN)),
            full((D_HIDDEN, 1)),
        ],
        out_specs=[
            pl.BlockSpec((3, N_CHANNEL, _BLKN),
                         lambda i: (0, 0, jnp.maximum(i - N // _BLKN, 0))),
            full((N_BATCH, D_HIDDEN)),
            full((N_BATCH, N_CHANNEL)),
            full((N_BATCH, 1)),
            full((N_BATCH, 1)),
        ],
        out_shape=[
            jax.ShapeDtypeStruct((3, N_CHANNEL, N), jnp.float32),
            jax.ShapeDtypeStruct((N_BATCH, D_HIDDEN), jnp.float32),
            jax.ShapeDtypeStruct((N_BATCH, N_CHANNEL), jnp.float32),
            jax.ShapeDtypeStruct((N_BATCH, 1), jnp.int32),
            jax.ShapeDtypeStruct((N_BATCH, 1), jnp.int32),
        ],
        scratch_shapes=[
            pltpu.VMEM((3, N_CHANNEL, N_BATCH), jnp.float32),
            pltpu.VMEM((3, N_CHANNEL, N_BATCH), jnp.float32),
            pltpu.VMEM((N_BATCH, 1), jnp.int32),
            pltpu.VMEM((N_BATCH, 1), jnp.int32),
            pltpu.VMEM((64, N_BATCH), jnp.float32),
            pltpu.VMEM((3, N_CHANNEL, N), jnp.float32),
        ],
    )(zt, blk_nat, bat_nat, sigma_c, W1, b1c, W2, b2c)

    ngrid3 = N // _BLKH
    h_out = pl.pallas_call(
        _happly_kernel,
        grid=(ngrid3,),
        in_specs=[
            pl.BlockSpec((_BLKH, D_HIDDEN), lambda i: (i, 0)),
            full((N_BATCH, 1)),
            full((N_BATCH, 1)),
            full((N_BATCH, D_HIDDEN)),
            full((1, D_HIDDEN)),
            full((1, D_HIDDEN)),
        ],
        out_specs=pl.BlockSpec((_BLKH, D_HIDDEN), lambda i: (i, 0)),
        out_shape=jax.ShapeDtypeStruct((N, D_HIDDEN), jnp.float32),
    )(H, usc, usnc, h2, gamma_r, beta_r)

    return (h_out, jnp.transpose(zo_t, (2, 1, 0)),
            resc.reshape(N_BATCH, N_CHANNEL, 1))


# speedup vs baseline: 157.0074x; 1.0014x over previous
import functools

import jax
import jax.numpy as jnp
from jax import lax
from jax.experimental import pallas as pl
from jax.experimental.pallas import tpu as pltpu

N = 65536
N_BLOCKS = 4096
N_BATCH = 16
D_HIDDEN = 128
N_CHANNEL = 14
N_RBF = 16
CUTOFF = 7.0

_BLKN = 32768
_BLKH = 16384

_DN_STD = (((1,), (0,)), ((), ()))
_DN_T0 = (((0,), (0,)), ((), ()))
_DN_T1 = (((1,), (1,)), ((), ()))


def _dot(a, b, dn=_DN_STD):
    return lax.dot_general(a, b, dn, preferred_element_type=jnp.float32)


def _silu(x):
    return x / (1.0 + jnp.exp(-x))


def _eye16():
    r16 = lax.broadcasted_iota(jnp.int32, (N_BATCH, N_BATCH), 0)
    c16 = lax.broadcasted_iota(jnp.int32, (N_BATCH, N_BATCH), 1)
    return jnp.where(r16 == c16, jnp.float32(1.0), jnp.float32(0.0))


def _onehot_t(base, blkn, usc, usnc):
    io1 = lax.broadcasted_iota(jnp.int32, (N_BATCH, blkn), 1) + base
    return jnp.where(jnp.logical_and(io1 >= usc, io1 < usnc),
                     jnp.float32(1.0), jnp.float32(0.0))


def _zpass_kernel(zt_ref, blk_ref, bat_ref, sigma_ref, w1_ref, b1_ref,
                  w2_ref, b2_ref,
                  zo_ref, h2_ref, resc_ref, usc_ref, usnc_ref,
                  s1_acc, s2_acc, usc_s, usnc_s, ztab_s, zcache, *, ng):
    i = pl.program_id(0)

    @pl.when(i == 0)
    def _init():
        blk = blk_ref[...]
        bat = bat_ref[...]
        starts = []
        for b in range(N_BATCH + 1):
            bs_b = jnp.sum((bat < b).astype(jnp.int32))
            starts.append(jnp.sum((blk < bs_b).astype(jnp.int32)))
        io_c = lax.broadcasted_iota(jnp.int32, (N_BATCH, 1), 0)
        usc = jnp.zeros((N_BATCH, 1), jnp.int32)
        usnc = jnp.zeros((N_BATCH, 1), jnp.int32)
        for b in range(N_BATCH):
            usc = jnp.where(io_c == b, starts[b], usc)
            usnc = jnp.where(io_c == b, starts[b + 1], usnc)
        usc_ref[...] = usc
        usnc_ref[...] = usnc
        usc_s[...] = usc
        usnc_s[...] = usnc
        s1_acc[...] = jnp.zeros_like(s1_acc)
        s2_acc[...] = jnp.zeros_like(s2_acc)
        ztab_s[...] = jnp.zeros_like(ztab_s)

    @pl.when(i < ng)
    def _stats():
        oh = _onehot_t(i * _BLKN, _BLKN, usc_s[...], usnc_s[...])
        z3 = zt_ref[...]
        zcache[:, :, pl.ds(i * _BLKN, _BLKN)] = z3
        for a in range(3):
            za = z3[a]
            s1_acc[a, :, :] += _dot(za, oh, _DN_T1)
            s2_acc[a, :, :] += _dot(za * za, oh, _DN_T1)

    @pl.when(i == ng - 1)
    def _middle():
        s1a = s1_acc[...]
        s2a = s2_acc[...]
        cnt = (usnc_s[...] - usc_s[...]).astype(jnp.float32)
        cnt_r = _dot(cnt, _eye16(), _DN_T0)
        cntc = jnp.maximum(cnt_r, 1.0)
        zc3 = s1a / cntc
        e = s2a - s1a * s1a / cntc
        t2 = e[0] + e[1] + e[2]
        denom = jnp.maximum(3.0 * cnt_r - 1.0, 1.0)
        std = jnp.sqrt(t2 / denom)
        resc_t = sigma_ref[...] / std
        resc_ref[...] = _dot(_eye16(), resc_t, _DN_T1)
        resc_safe = jnp.minimum(resc_t, jnp.float32(1e30))
        for a in range(3):
            ztab_s[16 * a:16 * a + N_CHANNEL, :] = zc3[a]
        ztab_s[48:48 + N_CHANNEL, :] = resc_safe
        nf = N_CHANNEL * N_RBF
        ce = lax.broadcasted_iota(jnp.int32, (nf, N_CHANNEL), 0)
        je = lax.broadcasted_iota(jnp.int32, (nf, N_CHANNEL), 1)
        exp_m = jnp.where(
            jnp.logical_and(ce >= N_RBF * je, ce < N_RBF * je + N_RBF),
            jnp.float32(1.0), jnp.float32(0.0))
        dsc = _dot(exp_m, resc_safe / CUTOFF)
        p = 5.0
        ea = -(p + 1.0) * (p + 2.0) / 2.0
        eb = p * (p + 2.0)
        ec = -p * (p + 1.0) / 2.0
        d4 = (dsc * dsc) * (dsc * dsc)
        env = 1.0 / dsc + ea * d4 + eb * d4 * dsc + ec * d4 * dsc * dsc
        env = jnp.where(dsc < 1.0, env, 0.0)
        f_io = lax.broadcasted_iota(jnp.int32, (nf, 1), 0)
        step = jnp.float32(1.0 / (N_RBF - 1))
        offs = (f_io % N_RBF).astype(jnp.float32) * step
        coeff = -0.5 / (step * step)
        gauss = jnp.exp(coeff * (dsc - offs) * (dsc - offs))
        rbf = env * gauss
        h1 = _silu(_dot(w1_ref[...], rbf) + b1_ref[...])
        h2 = _silu(_dot(w2_ref[...], h1) + b2_ref[...])
        h2_ref[...] = _dot(_eye16(), h2, _DN_T1)

    @pl.when(i >= ng)
    def _zapply():
        j = i - ng
        oh = _onehot_t(j * _BLKN, _BLKN, usc_s[...], usnc_s[...])
        g = _dot(ztab_s[...], oh)
        rg = g[48:48 + N_CHANNEL, :]
        z3 = zcache[:, :, pl.ds(j * _BLKN, _BLKN)]
        for a in range(3):
            zcg = g[16 * a:16 * a + N_CHANNEL, :]
            zo_ref[a, :, :] = zcg + (z3[a] - zcg) * rg


def _happly_kernel(h_ref, usc_ref, usnc_ref, h2_ref, gamma_ref, beta_ref,
                   ho_ref):
    i = pl.program_id(0)
    oh = _onehot_t(i * _BLKH, _BLKH, usc_ref[...], usnc_ref[...])
    hg = _dot(oh, h2_ref[...], _DN_T0)
    hn = h_ref[...] + hg
    mu = jnp.mean(hn, axis=1, keepdims=True)
    df = hn - mu
    v = jnp.mean(df * df, axis=1, keepdims=True)
    ho_ref[...] = df * lax.rsqrt(v + 1e-5) * gamma_ref[...] + beta_ref[...]


@jax.jit
def kernel(H, Z, block_id, batch_id, sigma, W1, b1, W2, b2, gamma, beta):
    zt = jnp.transpose(Z, (2, 1, 0))
    blk_nat = block_id.reshape(N // 128, 128).astype(jnp.int32)
    bat_nat = batch_id.reshape(N_BLOCKS // 128, 128).astype(jnp.int32)
    sigma_c = sigma.reshape(N_CHANNEL, 1)
    b1c = b1.reshape(D_HIDDEN, 1)
    b2c = b2.reshape(D_HIDDEN, 1)
    gamma_r = gamma.reshape(1, D_HIDDEN)
    beta_r = beta.reshape(1, D_HIDDEN)

    full = lambda shape: pl.BlockSpec(shape, lambda i: tuple(0 for _ in shape))
    ng = N // _BLKN
    zo_t, h2, resc, usc, usnc = pl.pallas_call(
        functools.partial(_zpass_kernel, ng=ng),
        grid=(2 * ng,),
        in_specs=[
            pl.BlockSpec((3, N_CHANNEL, _BLKN),
                         lambda i: (0, 0, jnp.minimum(i, N // _BLKN - 1))),
            full((N // 128, 128)),
            full((N_BLOCKS // 128, 128)),
            full((N_CHANNEL, 1)),
            full((D_HIDDEN, N_CHANNEL * N_RBF)),
            full((D_HIDDEN, 1)),
            full((D_HIDDEN, D_HIDDEN)),
            full((D_HIDDEN, 1)),
        ],
        out_specs=[
            pl.BlockSpec((3, N_CHANNEL, _BLKN),
                         lambda i: (0, 0, jnp.maximum(i - N // _BLKN, 0))),
            full((N_BATCH, D_HIDDEN)),
            full((N_BATCH, N_CHANNEL)),
            full((N_BATCH, 1)),
            full((N_BATCH, 1)),
        ],
        out_shape=[
            jax.ShapeDtypeStruct((3, N_CHANNEL, N), jnp.float32),
            jax.ShapeDtypeStruct((N_BATCH, D_HIDDEN), jnp.float32),
            jax.ShapeDtypeStruct((N_BATCH, N_CHANNEL), jnp.float32),
            jax.ShapeDtypeStruct((N_BATCH, 1), jnp.int32),
            jax.ShapeDtypeStruct((N_BATCH, 1), jnp.int32),
        ],
        scratch_shapes=[
            pltpu.VMEM((3, N_CHANNEL, N_BATCH), jnp.float32),
            pltpu.VMEM((3, N_CHANNEL, N_BATCH), jnp.float32),
            pltpu.VMEM((N_BATCH, 1), jnp.int32),
            pltpu.VMEM((N_BATCH, 1), jnp.int32),
            pltpu.VMEM((64, N_BATCH), jnp.float32),
            pltpu.VMEM((3, N_CHANNEL, N), jnp.float32),
        ],
    )(zt, blk_nat, bat_nat, sigma_c, W1, b1c, W2, b2c)

    ngrid3 = N // _BLKH
    h_out = pl.pallas_call(
        _happly_kernel,
        grid=(ngrid3,),
        in_specs=[
            pl.BlockSpec((_BLKH, D_HIDDEN), lambda i: (i, 0)),
            full((N_BATCH, 1)),
            full((N_BATCH, 1)),
            full((N_BATCH, D_HIDDEN)),
            full((1, D_HIDDEN)),
            full((1, D_HIDDEN)),
        ],
        out_specs=pl.BlockSpec((_BLKH, D_HIDDEN), lambda i: (i, 0)),
        out_shape=jax.ShapeDtypeStruct((N, D_HIDDEN), jnp.float32),
    )(H, usc, usnc, h2, gamma_r, beta_r)

    return (h_out, jnp.transpose(zo_t, (2, 1, 0)),
            resc.reshape(N_BATCH, N_CHANNEL, 1))
